# baseline (speedup 1.0000x reference)
import functools

import jax
import jax.numpy as jnp
import numpy as np
from jax import lax
from jax.experimental import pallas as pl
from jax.experimental.pallas import tpu as pltpu
from jax.experimental.pallas import tpu_sc as plsc

_C = 16
_PC = 2 * _C
_CH = 125
_G = 8
_ZR = 48
_NB = 4


def _probe_consts():
    freqs = jnp.power(2.0, jnp.linspace(0.0, _C // 2 - 0.5, _C)).reshape(1, -1)
    freqs = freqs.astype(jnp.float32)
    pk = jax.random.key(42)
    kx, ky, kz, kp, kf = jax.random.split(pk, 5)
    x_w = jax.random.uniform(kx, (1, 1), dtype=jnp.float32)
    y_w = jax.random.uniform(ky, (1, 1), dtype=jnp.float32)
    z_w = jax.random.uniform(kz, (1, 1), dtype=jnp.float32)
    x_w = x_w / (x_w + y_w + z_w)
    y_w = y_w / (x_w + y_w + z_w)
    z_w = z_w / (x_w + y_w + z_w)
    phase = jax.random.uniform(kp, (1, 1), dtype=jnp.float32) * 2.0 * np.pi
    freq_noise = jax.random.uniform(kf, (1, 1), dtype=jnp.float32) * 0.5 + 0.75
    freqs = freqs * freq_noise
    amp = 0.5 / freqs
    wvec = jnp.concatenate([x_w, y_w, z_w, phase], axis=1)
    return wvec, freqs, amp


def _probe_body(xs_ref, ys_ref, zs_ref, w_ref, f_ref, a_ref, out_ref):
    xw = w_ref[0, 0]
    yw = w_ref[0, 1]
    zw = w_ref[0, 2]
    ph = w_ref[0, 3]
    cs = xs_ref[...] * xw + ys_ref[...] * yw + zs_ref[...] * zw
    pin = cs * f_ref[0:1, :] + ph
    s = a_ref[0:1, :] * jnp.sin(pin)
    c = a_ref[0:1, :] * jnp.cos(pin)
    out_ref[...] = jnp.concatenate([s, c], axis=1)


def _make_probe(n, br=2000):
    grid = n // br
    return pl.pallas_call(
        _probe_body,
        grid=(grid,),
        in_specs=[
            pl.BlockSpec((br, 1), lambda i: (i, 0)),
            pl.BlockSpec((br, 1), lambda i: (i, 0)),
            pl.BlockSpec((br, 1), lambda i: (i, 0)),
            pl.BlockSpec((1, 4), lambda i: (0, 0)),
            pl.BlockSpec((1, _C), lambda i: (0, 0)),
            pl.BlockSpec((1, _C), lambda i: (0, 0)),
        ],
        out_specs=pl.BlockSpec((br, _PC), lambda i: (i, 0)),
        out_shape=jax.ShapeDtypeStruct((n, _PC), jnp.float32),
    )


def _comb_body(a0_ref, a1_ref, e0_ref, e1_ref, p_ref, m_ref, out_ref):
    ews = e0_ref[0] + e1_ref[0]
    acc = a0_ref[0] + a1_ref[0]
    out_ref[...] = (acc - ews * p_ref[...]) * (1.0 / m_ref[...])


def _make_combine(n, br=2000):
    grid = n // br
    return pl.pallas_call(
        _comb_body,
        grid=(grid,),
        in_specs=[
            pl.BlockSpec((1, br, _PC), lambda i: (0, i, 0)),
            pl.BlockSpec((1, br, _PC), lambda i: (1, i, 0)),
            pl.BlockSpec((1, br, 1), lambda i: (0, i, 0)),
            pl.BlockSpec((1, br, 1), lambda i: (1, i, 0)),
            pl.BlockSpec((br, _PC), lambda i: (i, 0)),
            pl.BlockSpec((br, 1), lambda i: (i, 0)),
        ],
        out_specs=pl.BlockSpec((br, _PC), lambda i: (i, 0)),
        out_shape=jax.ShapeDtypeStruct((n, _PC), jnp.float32),
    )


def _make_sc_scatter(n, rows_per_w):
    info = plsc.get_sparse_core_info()
    nc, ns, nl = info.num_cores, info.num_subcores, info.num_lanes
    nw = nc * ns
    blk = rows_per_w // _G
    npt = -(-(n // ns) // _ZR) * _ZR
    n_pad = npt * ns
    nfull = (_CH // nl) * nl
    mesh = plsc.VectorSubcoreMesh(core_axis_name="c", subcore_axis_name="s")

    @functools.partial(
        pl.kernel,
        out_type=(jax.ShapeDtypeStruct((nc, n_pad, _PC), jnp.float32),
                  jax.ShapeDtypeStruct((nc, n_pad), jnp.float32)),
        mesh=mesh,
        scratch_types=[
            pltpu.VMEM((_G, _CH), jnp.int32),
            pltpu.VMEM((_G, _CH), jnp.int32),
            pltpu.VMEM((_G, _CH), jnp.float32),
            [pltpu.VMEM((_CH, _PC), jnp.float32) for _ in range(_NB)],
            pltpu.VMEM((_ZR, _PC), jnp.float32),
            pltpu.VMEM((_ZR,), jnp.float32),
            pltpu.VMEM_SHARED((n_pad, _PC), jnp.float32),
            pltpu.VMEM_SHARED((n_pad,), jnp.float32),
            [pltpu.SemaphoreType.DMA for _ in range(_NB)],
            [pltpu.SemaphoreType.DMA for _ in range(_NB)],
            pltpu.SemaphoreType.DMA,
        ],
        compiler_params=pltpu.CompilerParams(
            use_tc_tiling_on_sc=False, needs_layout_passes=False),
    )
    def sc_scatter(probe_hbm, src_hbm, dst_hbm, w_hbm, acc_hbm, ews_hbm,
                   sbuf, dbuf, wbuf, rows, zbuf, zbuf1, acc, ews,
                   gsem, ssem, esem):
        cid = lax.axis_index("c")
        sid = lax.axis_index("s")
        wid = sid * nc + cid

        def _zfill(i, _):
            zbuf[i, pl.ds(0, nl)] = jnp.zeros((nl,), jnp.float32)
            zbuf[i, pl.ds(nl, nl)] = jnp.zeros((nl,), jnp.float32)
            return 0

        lax.fori_loop(0, _ZR, _zfill, 0)

        def _zfill1(i, _):
            zbuf1[pl.ds(i * nl, nl)] = jnp.zeros((nl,), jnp.float32)
            return 0

        lax.fori_loop(0, _ZR // nl, _zfill1, 0)

        def _zcopy(t, _):
            pltpu.sync_copy(zbuf, acc.at[pl.ds(sid * npt + t * _ZR, _ZR)])
            pltpu.sync_copy(zbuf1, ews.at[pl.ds(sid * npt + t * _ZR, _ZR)])
            return 0

        lax.fori_loop(0, npt // _ZR, _zcopy, 0)
        plsc.subcore_barrier()

        wbase = wid * rows_per_w

        def _scale(k, buf):
            def _group(g, _):
                wv = wbuf[k, pl.ds(g * nl, nl)]
                for i in range(nl):
                    r = g * nl + i
                    wb = jnp.full((nl,), wv[i], jnp.float32)
                    buf[r, pl.ds(0, nl)] = buf[r, pl.ds(0, nl)] * wb
                    buf[r, pl.ds(nl, nl)] = buf[r, pl.ds(nl, nl)] * wb
                return 0

            lax.fori_loop(0, _CH // nl, _group, 0)
            wv = wbuf[k, pl.ds(_CH - nl, nl)]
            for i in range(nfull - (_CH - nl), nl):
                r = _CH - nl + i
                wb = jnp.full((nl,), wv[i], jnp.float32)
                buf[r, pl.ds(0, nl)] = buf[r, pl.ds(0, nl)] * wb
                buf[r, pl.ds(nl, nl)] = buf[r, pl.ds(nl, nl)] * wb

        def _block(t, _):
            @pl.when(t > 0)
            def _():
                pltpu.make_async_copy(
                    rows[2], acc.at[dbuf.at[_G - 2]], ssem[2]).wait()
                pltpu.make_async_copy(
                    rows[3], acc.at[dbuf.at[_G - 1]], ssem[3]).wait()

            off = wbase + t * _G
            pltpu.sync_copy(src_hbm.at[pl.ds(off, _G)], sbuf)
            pltpu.sync_copy(dst_hbm.at[pl.ds(off, _G)], dbuf)
            pltpu.sync_copy(w_hbm.at[pl.ds(off, _G)], wbuf)

            edescs = []
            for u in range(_G):
                edescs.append(pltpu.async_copy(
                    wbuf.at[u], ews.at[sbuf.at[u]], esem, add=True))

            gdescs = {}
            sdescs = {}

            def _issue_g(u):
                gdescs[u] = pltpu.async_copy(
                    probe_hbm.at[sbuf.at[u]], rows[u % _NB], gsem[u % _NB])

            _issue_g(0)
            _issue_g(1)
            for u in range(_G):
                b = u % _NB
                gdescs[u].wait()
                _scale(u, rows[b])
                sdescs[u] = pltpu.async_copy(
                    rows[b], acc.at[dbuf.at[u]], ssem[b], add=True)
                if u + 2 < _G:
                    if u - 2 >= 0:
                        sdescs[u - 2].wait()
                    _issue_g(u + 2)
                else:
                    sdescs[u - 2].wait()
            for d in edescs:
                d.wait()
            return 0

        lax.fori_loop(0, blk, _block, 0)
        pltpu.make_async_copy(rows[2], acc.at[dbuf.at[_G - 2]], ssem[2]).wait()
        pltpu.make_async_copy(rows[3], acc.at[dbuf.at[_G - 1]], ssem[3]).wait()
        plsc.subcore_barrier()

        pltpu.sync_copy(acc.at[pl.ds(sid * npt, npt)],
                        acc_hbm.at[cid, pl.ds(sid * npt, npt)])
        pltpu.sync_copy(ews.at[pl.ds(sid * npt, npt)],
                        ews_hbm.at[cid, pl.ds(sid * npt, npt)])

    return sc_scatter


def kernel(x, edge_index, edge_width, vertex_mass):
    n = x.shape[0]
    e = edge_index.shape[1]
    src2 = edge_index[0].astype(jnp.int32).reshape(e // _CH, _CH)
    dst2 = edge_index[1].astype(jnp.int32).reshape(e // _CH, _CH)
    w2 = edge_width.astype(jnp.float32).reshape(e // _CH, _CH)

    wvec, freqs, amp = _probe_consts()
    probe = _make_probe(n)(
        x[:, 0:1], x[:, 1:2], x[:, 2:3], wvec, freqs, amp)

    nw = 32
    accs, ews = _make_sc_scatter(n, (e // _CH) // nw)(probe, src2, dst2, w2)

    ews3 = ews.reshape(ews.shape[0], ews.shape[1], 1)
    out = _make_combine(n)(accs, accs, ews3, ews3, probe, vertex_mass)
    return out


# speedup vs baseline: 1.3051x; 1.3051x over previous
import functools

import jax
import jax.numpy as jnp
import numpy as np
from jax import lax
from jax.experimental import pallas as pl
from jax.experimental.pallas import tpu as pltpu
from jax.experimental.pallas import tpu_sc as plsc

_C = 16
_PC = 2 * _C
_CH = 125
_G = 8
_ZR = 48
_NB = 4


def _probe_consts():
    freqs = jnp.power(2.0, jnp.linspace(0.0, _C // 2 - 0.5, _C)).reshape(1, -1)
    freqs = freqs.astype(jnp.float32)
    pk = jax.random.key(42)
    kx, ky, kz, kp, kf = jax.random.split(pk, 5)
    x_w = jax.random.uniform(kx, (1, 1), dtype=jnp.float32)
    y_w = jax.random.uniform(ky, (1, 1), dtype=jnp.float32)
    z_w = jax.random.uniform(kz, (1, 1), dtype=jnp.float32)
    x_w = x_w / (x_w + y_w + z_w)
    y_w = y_w / (x_w + y_w + z_w)
    z_w = z_w / (x_w + y_w + z_w)
    phase = jax.random.uniform(kp, (1, 1), dtype=jnp.float32) * 2.0 * np.pi
    freq_noise = jax.random.uniform(kf, (1, 1), dtype=jnp.float32) * 0.5 + 0.75
    freqs = freqs * freq_noise
    amp = 0.5 / freqs
    wvec = jnp.concatenate([x_w, y_w, z_w, phase], axis=1)
    return wvec, freqs, amp


_RPR = 128 // _PC


def _probe_body(x_ref, w_ref, f_ref, a_ref, sel_ref, out_ref):
    xw = w_ref[0, 0]
    yw = w_ref[0, 1]
    zw = w_ref[0, 2]
    ph = w_ref[0, 3]
    xv = x_ref[...]
    n4 = xv.shape[0]

    def _cols(start):
        idx = jnp.broadcast_to(
            jnp.arange(start, 3 * _RPR, 3, dtype=jnp.int32)[None, :],
            (n4, _RPR))
        return jnp.take_along_axis(xv, idx, axis=1)

    cs4 = _cols(0) * xw + _cols(1) * yw + _cols(2) * zw
    sel2 = jnp.broadcast_to(sel_ref[0:1, :], (n4, 128))
    csr = jnp.take_along_axis(cs4, sel2, axis=1)
    pin = csr * f_ref[0:1, :] + ph
    lane = jax.lax.broadcasted_iota(jnp.int32, (1, 128), 1)
    is_sin = (lane % _PC) < _C
    out_ref[...] = a_ref[0:1, :] * jnp.where(is_sin, jnp.sin(pin),
                                             jnp.cos(pin))


def _make_probe(n):
    n4 = n // _RPR
    return pl.pallas_call(
        _probe_body,
        grid=(1,),
        in_specs=[
            pl.BlockSpec((n4, 3 * _RPR), lambda i: (0, 0)),
            pl.BlockSpec((1, 4), lambda i: (0, 0)),
            pl.BlockSpec((1, 128), lambda i: (0, 0)),
            pl.BlockSpec((1, 128), lambda i: (0, 0)),
            pl.BlockSpec((1, 128), lambda i: (0, 0)),
        ],
        out_specs=pl.BlockSpec((n4, 128), lambda i: (0, 0)),
        out_shape=jax.ShapeDtypeStruct((n4, 128), jnp.float32),
    )


def _comb_body(n4, a0_ref, a1_ref, e0_ref, e1_ref, p_ref, m_ref, sel_ref,
               out_ref):
    ews4 = e0_ref[0, :n4] + e1_ref[0, :n4]
    sel2 = jnp.broadcast_to(sel_ref[0:1, :], (n4, 128))
    ews = jnp.take_along_axis(ews4, sel2, axis=1)
    m = jnp.take_along_axis(m_ref[...], sel2, axis=1)
    acc = a0_ref[0, :n4] + a1_ref[0, :n4]
    out_ref[...] = (acc - ews * p_ref[...]) * (1.0 / m)


def _make_combine(n, n_pad):
    n4 = n // _RPR
    np4 = n_pad // _RPR
    return pl.pallas_call(
        functools.partial(_comb_body, n4),
        grid=(1,),
        in_specs=[
            pl.BlockSpec((1, np4, 128), lambda i: (0, 0, 0)),
            pl.BlockSpec((1, np4, 128), lambda i: (1, 0, 0)),
            pl.BlockSpec((1, np4, _RPR), lambda i: (0, 0, 0)),
            pl.BlockSpec((1, np4, _RPR), lambda i: (1, 0, 0)),
            pl.BlockSpec((n4, 128), lambda i: (0, 0)),
            pl.BlockSpec((n4, _RPR), lambda i: (0, 0)),
            pl.BlockSpec((1, 128), lambda i: (0, 0)),
        ],
        out_specs=pl.BlockSpec((n4, 128), lambda i: (0, 0)),
        out_shape=jax.ShapeDtypeStruct((n4, 128), jnp.float32),
    )


def _make_sc_scatter(n, rows_per_w):
    info = plsc.get_sparse_core_info()
    nc, ns, nl = info.num_cores, info.num_subcores, info.num_lanes
    nw = nc * ns
    blk = rows_per_w // _G
    npt = -(-(n // ns) // _ZR) * _ZR
    n_pad = npt * ns
    nfull = (_CH // nl) * nl
    mesh = plsc.VectorSubcoreMesh(core_axis_name="c", subcore_axis_name="s")

    @functools.partial(
        pl.kernel,
        out_type=(jax.ShapeDtypeStruct((nc, n_pad, _PC), jnp.float32),
                  jax.ShapeDtypeStruct((nc, n_pad), jnp.float32)),
        mesh=mesh,
        scratch_types=[
            pltpu.VMEM((_G, _CH), jnp.int32),
            pltpu.VMEM((_G, _CH), jnp.int32),
            pltpu.VMEM((_G, _CH), jnp.float32),
            [pltpu.VMEM((_CH, _PC), jnp.float32) for _ in range(_NB)],
            pltpu.VMEM((_ZR, _PC), jnp.float32),
            pltpu.VMEM((_ZR,), jnp.float32),
            pltpu.VMEM_SHARED((n_pad, _PC), jnp.float32),
            pltpu.VMEM_SHARED((n_pad,), jnp.float32),
            [pltpu.SemaphoreType.DMA for _ in range(_NB)],
            [pltpu.SemaphoreType.DMA for _ in range(_NB)],
            pltpu.SemaphoreType.DMA,
        ],
        compiler_params=pltpu.CompilerParams(
            use_tc_tiling_on_sc=False, needs_layout_passes=False),
    )
    def sc_scatter(probe_hbm, src_hbm, dst_hbm, w_hbm, acc_hbm, ews_hbm,
                   sbuf, dbuf, wbuf, rows, zbuf, zbuf1, acc, ews,
                   gsem, ssem, esem):
        cid = lax.axis_index("c")
        sid = lax.axis_index("s")
        wid = sid * nc + cid

        def _zfill(i, _):
            zbuf[i, pl.ds(0, nl)] = jnp.zeros((nl,), jnp.float32)
            zbuf[i, pl.ds(nl, nl)] = jnp.zeros((nl,), jnp.float32)
            return 0

        lax.fori_loop(0, _ZR, _zfill, 0)

        def _zfill1(i, _):
            zbuf1[pl.ds(i * nl, nl)] = jnp.zeros((nl,), jnp.float32)
            return 0

        lax.fori_loop(0, _ZR // nl, _zfill1, 0)

        def _zcopy(t, _):
            pltpu.sync_copy(zbuf, acc.at[pl.ds(sid * npt + t * _ZR, _ZR)])
            pltpu.sync_copy(zbuf1, ews.at[pl.ds(sid * npt + t * _ZR, _ZR)])
            return 0

        lax.fori_loop(0, npt // _ZR, _zcopy, 0)
        plsc.subcore_barrier()

        wbase = wid * rows_per_w

        def _scale(k, buf):
            def _group(g, _):
                wv = wbuf[k, pl.ds(g * nl, nl)]
                for i in range(nl):
                    r = g * nl + i
                    wb = jnp.full((nl,), wv[i], jnp.float32)
                    buf[r, pl.ds(0, nl)] = buf[r, pl.ds(0, nl)] * wb
                    buf[r, pl.ds(nl, nl)] = buf[r, pl.ds(nl, nl)] * wb
                return 0

            lax.fori_loop(0, _CH // nl, _group, 0)
            wv = wbuf[k, pl.ds(_CH - nl, nl)]
            for i in range(nfull - (_CH - nl), nl):
                r = _CH - nl + i
                wb = jnp.full((nl,), wv[i], jnp.float32)
                buf[r, pl.ds(0, nl)] = buf[r, pl.ds(0, nl)] * wb
                buf[r, pl.ds(nl, nl)] = buf[r, pl.ds(nl, nl)] * wb

        def _block(t, _):
            @pl.when(t > 0)
            def _():
                pltpu.make_async_copy(
                    rows[2], acc.at[dbuf.at[_G - 2]], ssem[2]).wait()
                pltpu.make_async_copy(
                    rows[3], acc.at[dbuf.at[_G - 1]], ssem[3]).wait()

            off = wbase + t * _G
            pltpu.sync_copy(src_hbm.at[pl.ds(off, _G)], sbuf)
            pltpu.sync_copy(dst_hbm.at[pl.ds(off, _G)], dbuf)
            pltpu.sync_copy(w_hbm.at[pl.ds(off, _G)], wbuf)

            edescs = []
            for u in range(_G):
                edescs.append(pltpu.async_copy(
                    wbuf.at[u], ews.at[sbuf.at[u]], esem, add=True))

            gdescs = {}
            sdescs = {}

            def _issue_g(u):
                gdescs[u] = pltpu.async_copy(
                    probe_hbm.at[sbuf.at[u]], rows[u % _NB], gsem[u % _NB])

            _issue_g(0)
            _issue_g(1)
            for u in range(_G):
                b = u % _NB
                gdescs[u].wait()
                _scale(u, rows[b])
                sdescs[u] = pltpu.async_copy(
                    rows[b], acc.at[dbuf.at[u]], ssem[b], add=True)
                if u + 2 < _G:
                    if u - 2 >= 0:
                        sdescs[u - 2].wait()
                    _issue_g(u + 2)
                else:
                    sdescs[u - 2].wait()
            for d in edescs:
                d.wait()
            return 0

        lax.fori_loop(0, blk, _block, 0)
        pltpu.make_async_copy(rows[2], acc.at[dbuf.at[_G - 2]], ssem[2]).wait()
        pltpu.make_async_copy(rows[3], acc.at[dbuf.at[_G - 1]], ssem[3]).wait()
        plsc.subcore_barrier()

        pltpu.sync_copy(acc.at[pl.ds(sid * npt, npt)],
                        acc_hbm.at[cid, pl.ds(sid * npt, npt)])
        pltpu.sync_copy(ews.at[pl.ds(sid * npt, npt)],
                        ews_hbm.at[cid, pl.ds(sid * npt, npt)])

    return sc_scatter


def kernel(x, edge_index, edge_width, vertex_mass):
    n = x.shape[0]
    e = edge_index.shape[1]
    src2 = edge_index[0].astype(jnp.int32).reshape(e // _CH, _CH)
    dst2 = edge_index[1].astype(jnp.int32).reshape(e // _CH, _CH)
    w2 = edge_width.astype(jnp.float32).reshape(e // _CH, _CH)

    wvec, freqs, amp = _probe_consts()
    f2 = jnp.concatenate([freqs, freqs], axis=1)
    f128 = jnp.tile(f2, (1, _RPR))
    a128 = jnp.tile(jnp.concatenate([amp, amp], axis=1), (1, _RPR))
    sel = (jnp.arange(128, dtype=jnp.int32) // _PC).reshape(1, 128)

    xx = x.reshape(n // _RPR, 3 * _RPR)
    probe128 = _make_probe(n)(xx, wvec, f128, a128, sel)

    nw = 32
    accs, ews = _make_sc_scatter(n, (e // _CH) // nw)(
        probe128.reshape(n, _PC), src2, dst2, w2)

    n_pad = ews.shape[1]
    acc128 = accs.reshape(2, n_pad // _RPR, 128)
    e4 = ews.reshape(2, n_pad // _RPR, _RPR)
    m4 = vertex_mass.reshape(n // _RPR, _RPR)
    out128 = _make_combine(n, n_pad)(acc128, acc128, e4, e4, probe128, m4, sel)
    return out128.reshape(n, _PC)


# speedup vs baseline: 1.5183x; 1.1634x over previous
import functools

import jax
import jax.numpy as jnp
import numpy as np
from jax import lax
from jax.experimental import pallas as pl
from jax.experimental.pallas import tpu as pltpu
from jax.experimental.pallas import tpu_sc as plsc

_C = 16
_PC = 2 * _C
_CH = 128
_G = 8
_ZR = 48
_NB = 4


def _probe_consts():
    freqs = jnp.power(2.0, jnp.linspace(0.0, _C // 2 - 0.5, _C)).reshape(1, -1)
    freqs = freqs.astype(jnp.float32)
    pk = jax.random.key(42)
    kx, ky, kz, kp, kf = jax.random.split(pk, 5)
    x_w = jax.random.uniform(kx, (1, 1), dtype=jnp.float32)
    y_w = jax.random.uniform(ky, (1, 1), dtype=jnp.float32)
    z_w = jax.random.uniform(kz, (1, 1), dtype=jnp.float32)
    x_w = x_w / (x_w + y_w + z_w)
    y_w = y_w / (x_w + y_w + z_w)
    z_w = z_w / (x_w + y_w + z_w)
    phase = jax.random.uniform(kp, (1, 1), dtype=jnp.float32) * 2.0 * np.pi
    freq_noise = jax.random.uniform(kf, (1, 1), dtype=jnp.float32) * 0.5 + 0.75
    freqs = freqs * freq_noise
    amp = 0.5 / freqs
    wvec = jnp.concatenate([x_w, y_w, z_w, phase], axis=1)
    return wvec, freqs, amp


_RPR = 128 // _PC


def _probe_body(x_ref, w_ref, f_ref, a_ref, sel_ref, out_ref):
    xw = w_ref[0, 0]
    yw = w_ref[0, 1]
    zw = w_ref[0, 2]
    ph = w_ref[0, 3]
    xv = x_ref[...]
    n4 = xv.shape[0]

    def _cols(start):
        idx = jnp.broadcast_to(
            jnp.arange(start, 3 * _RPR, 3, dtype=jnp.int32)[None, :],
            (n4, _RPR))
        return jnp.take_along_axis(xv, idx, axis=1)

    cs4 = _cols(0) * xw + _cols(1) * yw + _cols(2) * zw
    sel2 = jnp.broadcast_to(sel_ref[0:1, :], (n4, 128))
    csr = jnp.take_along_axis(cs4, sel2, axis=1)
    pin = csr * f_ref[0:1, :] + ph
    lane = jax.lax.broadcasted_iota(jnp.int32, (1, 128), 1)
    is_sin = (lane % _PC) < _C
    out_ref[...] = a_ref[0:1, :] * jnp.where(is_sin, jnp.sin(pin),
                                             jnp.cos(pin))


def _make_probe(n):
    n4 = n // _RPR
    return pl.pallas_call(
        _probe_body,
        grid=(1,),
        in_specs=[
            pl.BlockSpec((n4, 3 * _RPR), lambda i: (0, 0)),
            pl.BlockSpec((1, 4), lambda i: (0, 0)),
            pl.BlockSpec((1, 128), lambda i: (0, 0)),
            pl.BlockSpec((1, 128), lambda i: (0, 0)),
            pl.BlockSpec((1, 128), lambda i: (0, 0)),
        ],
        out_specs=pl.BlockSpec((n4, 128), lambda i: (0, 0)),
        out_shape=jax.ShapeDtypeStruct((n4, 128), jnp.float32),
    )


def _comb_body(n4, a0_ref, a1_ref, e0_ref, e1_ref, p_ref, m_ref, sel_ref,
               out_ref):
    ews4 = e0_ref[0, :n4] + e1_ref[0, :n4]
    sel2 = jnp.broadcast_to(sel_ref[0:1, :], (n4, 128))
    ews = jnp.take_along_axis(ews4, sel2, axis=1)
    m = jnp.take_along_axis(m_ref[...], sel2, axis=1)
    acc = a0_ref[0, :n4] + a1_ref[0, :n4]
    out_ref[...] = (acc - ews * p_ref[...]) * (1.0 / m)


def _make_combine(n, n_pad):
    n4 = n // _RPR
    np4 = n_pad // _RPR
    return pl.pallas_call(
        functools.partial(_comb_body, n4),
        grid=(1,),
        in_specs=[
            pl.BlockSpec((1, np4, 128), lambda i: (0, 0, 0)),
            pl.BlockSpec((1, np4, 128), lambda i: (1, 0, 0)),
            pl.BlockSpec((1, np4, _RPR), lambda i: (0, 0, 0)),
            pl.BlockSpec((1, np4, _RPR), lambda i: (1, 0, 0)),
            pl.BlockSpec((n4, 128), lambda i: (0, 0)),
            pl.BlockSpec((n4, _RPR), lambda i: (0, 0)),
            pl.BlockSpec((1, 128), lambda i: (0, 0)),
        ],
        out_specs=pl.BlockSpec((n4, 128), lambda i: (0, 0)),
        out_shape=jax.ShapeDtypeStruct((n4, 128), jnp.float32),
    )


def _make_sc_scatter(n, e):
    info = plsc.get_sparse_core_info()
    nc, ns, nl = info.num_cores, info.num_subcores, info.num_lanes
    nw = nc * ns
    epw = e // nw
    bsz = _G * _CH
    blk = epw // bsz
    tail = epw - blk * bsz
    tail_chunks = [(o, min(_CH, tail - o)) for o in range(0, tail, _CH)]
    npt = -(-(n // ns) // _ZR) * _ZR
    n_pad = npt * ns
    mesh = plsc.VectorSubcoreMesh(core_axis_name="c", subcore_axis_name="s")

    @functools.partial(
        pl.kernel,
        out_type=(jax.ShapeDtypeStruct((nc, n_pad, _PC), jnp.float32),
                  jax.ShapeDtypeStruct((nc, n_pad), jnp.float32)),
        mesh=mesh,
        scratch_types=[
            pltpu.VMEM((bsz,), jnp.int32),
            pltpu.VMEM((bsz,), jnp.int32),
            pltpu.VMEM((bsz,), jnp.float32),
            [pltpu.VMEM((_CH, _PC), jnp.float32) for _ in range(_NB)],
            pltpu.VMEM((_ZR, _PC), jnp.float32),
            pltpu.VMEM((_ZR,), jnp.float32),
            pltpu.VMEM_SHARED((n_pad, _PC), jnp.float32),
            pltpu.VMEM_SHARED((n_pad,), jnp.float32),
            [pltpu.SemaphoreType.DMA for _ in range(_NB)],
            [pltpu.SemaphoreType.DMA for _ in range(_NB)],
            pltpu.SemaphoreType.DMA,
        ],
        compiler_params=pltpu.CompilerParams(
            use_tc_tiling_on_sc=False, needs_layout_passes=False),
    )
    def sc_scatter(probe_hbm, ei_hbm, w_hbm, acc_hbm, ews_hbm,
                   sbuf, dbuf, wbuf, rows, zbuf, zbuf1, acc, ews,
                   gsem, ssem, esem):
        cid = lax.axis_index("c")
        sid = lax.axis_index("s")
        wid = sid * nc + cid

        def _zfill(i, _):
            zbuf[i, pl.ds(0, nl)] = jnp.zeros((nl,), jnp.float32)
            zbuf[i, pl.ds(nl, nl)] = jnp.zeros((nl,), jnp.float32)
            return 0

        lax.fori_loop(0, _ZR, _zfill, 0)

        def _zfill1(i, _):
            zbuf1[pl.ds(i * nl, nl)] = jnp.zeros((nl,), jnp.float32)
            return 0

        lax.fori_loop(0, _ZR // nl, _zfill1, 0)

        def _zcopy(t, _):
            pltpu.sync_copy(zbuf, acc.at[pl.ds(sid * npt + t * _ZR, _ZR)])
            pltpu.sync_copy(zbuf1, ews.at[pl.ds(sid * npt + t * _ZR, _ZR)])
            return 0

        lax.fori_loop(0, npt // _ZR, _zcopy, 0)
        plsc.subcore_barrier()

        wbase = wid * epw

        def _scale(o, cnt, buf):
            def _group(g, _):
                wv = wbuf[pl.ds(o + g * nl, nl)]
                for i in range(nl):
                    wb = jnp.full((nl,), wv[i], jnp.float32)
                    buf[g * nl + i, pl.ds(0, nl)] = (
                        buf[g * nl + i, pl.ds(0, nl)] * wb)
                    buf[g * nl + i, pl.ds(nl, nl)] = (
                        buf[g * nl + i, pl.ds(nl, nl)] * wb)
                return 0

            lax.fori_loop(0, cnt // nl, _group, 0)

        def _run_chunks(chunks):
            edescs = []
            for (o, cnt) in chunks:
                edescs.append(pltpu.async_copy(
                    wbuf.at[pl.ds(o, cnt)], ews.at[sbuf.at[pl.ds(o, cnt)]],
                    esem, add=True))

            nch = len(chunks)
            gdescs = {}
            sdescs = {}

            def _issue_g(u):
                o, cnt = chunks[u]
                b = u % _NB
                dst = rows[b] if cnt == _CH else rows[b].at[pl.ds(0, cnt)]
                gdescs[u] = pltpu.async_copy(
                    probe_hbm.at[sbuf.at[pl.ds(o, cnt)]], dst, gsem[b])

            _issue_g(0)
            if nch > 1:
                _issue_g(1)
            for u in range(nch):
                o, cnt = chunks[u]
                b = u % _NB
                gdescs[u].wait()
                _scale(o, cnt, rows[b])
                src = rows[b] if cnt == _CH else rows[b].at[pl.ds(0, cnt)]
                sdescs[u] = pltpu.async_copy(
                    src, acc.at[dbuf.at[pl.ds(o, cnt)]], ssem[b], add=True)
                if u + 2 < nch:
                    if u - 2 >= 0:
                        sdescs[u - 2].wait()
                    _issue_g(u + 2)
                elif u - 2 >= 0:
                    sdescs[u - 2].wait()
            for d in edescs:
                d.wait()

        full_chunks = [(u * _CH, _CH) for u in range(_G)]

        def _drain_last_two(chunks):
            for u in (len(chunks) - 2, len(chunks) - 1):
                if u < 0:
                    continue
                o, cnt = chunks[u]
                b = u % _NB
                src = rows[b] if cnt == _CH else rows[b].at[pl.ds(0, cnt)]
                pltpu.make_async_copy(
                    src, acc.at[dbuf.at[pl.ds(o, cnt)]], ssem[b]).wait()

        def _block(t, _):
            @pl.when(t > 0)
            def _():
                _drain_last_two(full_chunks)

            off = wbase + t * bsz
            pltpu.sync_copy(ei_hbm.at[0, pl.ds(off, bsz)], sbuf)
            pltpu.sync_copy(ei_hbm.at[1, pl.ds(off, bsz)], dbuf)
            pltpu.sync_copy(w_hbm.at[pl.ds(off, bsz)], wbuf)
            _run_chunks(full_chunks)
            return 0

        lax.fori_loop(0, blk, _block, 0)
        _drain_last_two(full_chunks)
        if tail:
            toff = wbase + blk * bsz
            pltpu.sync_copy(ei_hbm.at[0, pl.ds(toff, tail)],
                            sbuf.at[pl.ds(0, tail)])
            pltpu.sync_copy(ei_hbm.at[1, pl.ds(toff, tail)],
                            dbuf.at[pl.ds(0, tail)])
            pltpu.sync_copy(w_hbm.at[pl.ds(toff, tail)],
                            wbuf.at[pl.ds(0, tail)])
            _run_chunks(tail_chunks)
            _drain_last_two(tail_chunks)
        plsc.subcore_barrier()

        pltpu.sync_copy(acc.at[pl.ds(sid * npt, npt)],
                        acc_hbm.at[cid, pl.ds(sid * npt, npt)])
        pltpu.sync_copy(ews.at[pl.ds(sid * npt, npt)],
                        ews_hbm.at[cid, pl.ds(sid * npt, npt)])

    return sc_scatter


def kernel(x, edge_index, edge_width, vertex_mass):
    n = x.shape[0]
    e = edge_index.shape[1]
    ei = edge_index.astype(jnp.int32)
    w1 = edge_width.astype(jnp.float32).reshape(e)

    wvec, freqs, amp = _probe_consts()
    f2 = jnp.concatenate([freqs, freqs], axis=1)
    f128 = jnp.tile(f2, (1, _RPR))
    a128 = jnp.tile(jnp.concatenate([amp, amp], axis=1), (1, _RPR))
    sel = (jnp.arange(128, dtype=jnp.int32) // _PC).reshape(1, 128)

    xx = x.reshape(n // _RPR, 3 * _RPR)
    probe128 = _make_probe(n)(xx, wvec, f128, a128, sel)

    accs, ews = _make_sc_scatter(n, e)(probe128.reshape(n, _PC), ei, w1)

    n_pad = ews.shape[1]
    acc128 = accs.reshape(2, n_pad // _RPR, 128)
    e4 = ews.reshape(2, n_pad // _RPR, _RPR)
    m4 = vertex_mass.reshape(n // _RPR, _RPR)
    out128 = _make_combine(n, n_pad)(acc128, acc128, e4, e4, probe128, m4, sel)
    return out128.reshape(n, _PC)


# speedup vs baseline: 1.5188x; 1.0003x over previous
import functools

import jax
import jax.numpy as jnp
import numpy as np
from jax import lax
from jax.experimental import pallas as pl
from jax.experimental.pallas import tpu as pltpu
from jax.experimental.pallas import tpu_sc as plsc

_C = 16
_PC = 2 * _C
_CH = 128
_G = 8
_ZR = 48
_NB = 4


def _probe_consts():
    freqs = jnp.power(2.0, jnp.linspace(0.0, _C // 2 - 0.5, _C)).reshape(1, -1)
    freqs = freqs.astype(jnp.float32)
    pk = jax.random.key(42)
    kx, ky, kz, kp, kf = jax.random.split(pk, 5)
    x_w = jax.random.uniform(kx, (1, 1), dtype=jnp.float32)
    y_w = jax.random.uniform(ky, (1, 1), dtype=jnp.float32)
    z_w = jax.random.uniform(kz, (1, 1), dtype=jnp.float32)
    x_w = x_w / (x_w + y_w + z_w)
    y_w = y_w / (x_w + y_w + z_w)
    z_w = z_w / (x_w + y_w + z_w)
    phase = jax.random.uniform(kp, (1, 1), dtype=jnp.float32) * 2.0 * np.pi
    freq_noise = jax.random.uniform(kf, (1, 1), dtype=jnp.float32) * 0.5 + 0.75
    freqs = freqs * freq_noise
    amp = 0.5 / freqs
    wvec = jnp.concatenate([x_w, y_w, z_w, phase], axis=1)
    return wvec, freqs, amp


_RPR = 128 // _PC


def _probe_body(x_ref, w_ref, f_ref, a_ref, sel_ref, out_ref):
    xw = w_ref[0, 0]
    yw = w_ref[0, 1]
    zw = w_ref[0, 2]
    ph = w_ref[0, 3]
    xv = x_ref[...]
    n4 = xv.shape[0]

    def _cols(start):
        idx = jnp.broadcast_to(
            jnp.arange(start, 3 * _RPR, 3, dtype=jnp.int32)[None, :],
            (n4, _RPR))
        return jnp.take_along_axis(xv, idx, axis=1)

    cs4 = _cols(0) * xw + _cols(1) * yw + _cols(2) * zw
    sel2 = jnp.broadcast_to(sel_ref[0:1, :], (n4, 128))
    csr = jnp.take_along_axis(cs4, sel2, axis=1)
    pin = csr * f_ref[0:1, :] + ph
    lane = jax.lax.broadcasted_iota(jnp.int32, (1, 128), 1)
    is_sin = (lane % _PC) < _C
    out_ref[...] = a_ref[0:1, :] * jnp.where(is_sin, jnp.sin(pin),
                                             jnp.cos(pin))


def _make_probe(n):
    n4 = n // _RPR
    return pl.pallas_call(
        _probe_body,
        grid=(1,),
        in_specs=[
            pl.BlockSpec((n4, 3 * _RPR), lambda i: (0, 0)),
            pl.BlockSpec((1, 4), lambda i: (0, 0)),
            pl.BlockSpec((1, 128), lambda i: (0, 0)),
            pl.BlockSpec((1, 128), lambda i: (0, 0)),
            pl.BlockSpec((1, 128), lambda i: (0, 0)),
        ],
        out_specs=pl.BlockSpec((n4, 128), lambda i: (0, 0)),
        out_shape=jax.ShapeDtypeStruct((n4, 128), jnp.float32),
    )


def _comb_body(n4, a0_ref, a1_ref, e0_ref, e1_ref, p_ref, m_ref, sel_ref,
               out_ref):
    ews4 = e0_ref[0, :n4] + e1_ref[0, :n4]
    sel2 = jnp.broadcast_to(sel_ref[0:1, :], (n4, 128))
    ews = jnp.take_along_axis(ews4, sel2, axis=1)
    m = jnp.take_along_axis(m_ref[...], sel2, axis=1)
    acc = a0_ref[0, :n4] + a1_ref[0, :n4]
    out_ref[...] = (acc - ews * p_ref[...]) * (1.0 / m)


def _make_combine(n, n_pad):
    n4 = n // _RPR
    np4 = n_pad // _RPR
    return pl.pallas_call(
        functools.partial(_comb_body, n4),
        grid=(1,),
        in_specs=[
            pl.BlockSpec((1, np4, 128), lambda i: (0, 0, 0)),
            pl.BlockSpec((1, np4, 128), lambda i: (1, 0, 0)),
            pl.BlockSpec((1, np4, _RPR), lambda i: (0, 0, 0)),
            pl.BlockSpec((1, np4, _RPR), lambda i: (1, 0, 0)),
            pl.BlockSpec((n4, 128), lambda i: (0, 0)),
            pl.BlockSpec((n4, _RPR), lambda i: (0, 0)),
            pl.BlockSpec((1, 128), lambda i: (0, 0)),
        ],
        out_specs=pl.BlockSpec((n4, 128), lambda i: (0, 0)),
        out_shape=jax.ShapeDtypeStruct((n4, 128), jnp.float32),
    )


def _make_sc_scatter(n, e):
    info = plsc.get_sparse_core_info()
    nc, ns, nl = info.num_cores, info.num_subcores, info.num_lanes
    nw = nc * ns
    epw = e // nw
    bsz = _G * _CH
    blk = epw // bsz
    tail = epw - blk * bsz
    tail_chunks = [(o, min(_CH, tail - o)) for o in range(0, tail, _CH)]
    npt = -(-(n // ns) // _ZR) * _ZR
    n_pad = npt * ns
    mesh = plsc.VectorSubcoreMesh(core_axis_name="c", subcore_axis_name="s")

    @functools.partial(
        pl.kernel,
        out_type=(jax.ShapeDtypeStruct((nc, n_pad, _PC), jnp.float32),
                  jax.ShapeDtypeStruct((nc, n_pad), jnp.float32)),
        mesh=mesh,
        scratch_types=[
            pltpu.VMEM((bsz,), jnp.int32),
            pltpu.VMEM((bsz,), jnp.int32),
            pltpu.VMEM((bsz,), jnp.float32),
            [pltpu.VMEM((_CH, _PC), jnp.float32) for _ in range(_NB)],
            pltpu.VMEM((_ZR, _PC), jnp.float32),
            pltpu.VMEM((_ZR,), jnp.float32),
            pltpu.VMEM_SHARED((n_pad, _PC), jnp.float32),
            pltpu.VMEM_SHARED((n_pad,), jnp.float32),
            [pltpu.SemaphoreType.DMA for _ in range(_NB)],
            [pltpu.SemaphoreType.DMA for _ in range(_NB)],
            pltpu.SemaphoreType.DMA,
        ],
        compiler_params=pltpu.CompilerParams(
            use_tc_tiling_on_sc=False, needs_layout_passes=False),
    )
    def sc_scatter(probe_hbm, ei_hbm, w_hbm, acc_hbm, ews_hbm,
                   sbuf, dbuf, wbuf, rows, zbuf, zbuf1, acc, ews,
                   gsem, ssem, esem):
        cid = lax.axis_index("c")
        sid = lax.axis_index("s")
        wid = sid * nc + cid

        def _zfill(i, _):
            zbuf[i, pl.ds(0, nl)] = jnp.zeros((nl,), jnp.float32)
            zbuf[i, pl.ds(nl, nl)] = jnp.zeros((nl,), jnp.float32)
            return 0

        lax.fori_loop(0, _ZR, _zfill, 0)

        def _zfill1(i, _):
            zbuf1[pl.ds(i * nl, nl)] = jnp.zeros((nl,), jnp.float32)
            return 0

        lax.fori_loop(0, _ZR // nl, _zfill1, 0)

        def _zcopy(t, _):
            pltpu.sync_copy(zbuf, acc.at[pl.ds(sid * npt + t * _ZR, _ZR)])
            pltpu.sync_copy(zbuf1, ews.at[pl.ds(sid * npt + t * _ZR, _ZR)])
            return 0

        lax.fori_loop(0, npt // _ZR, _zcopy, 0)
        plsc.subcore_barrier()

        wbase = wid * epw

        def _scale(o, cnt, buf):
            def _group(g, _):
                wv = wbuf[pl.ds(o + g * nl, nl)]
                for i in range(nl):
                    wb = jnp.full((nl,), wv[i], jnp.float32)
                    buf[g * nl + i, pl.ds(0, nl)] = (
                        buf[g * nl + i, pl.ds(0, nl)] * wb)
                    buf[g * nl + i, pl.ds(nl, nl)] = (
                        buf[g * nl + i, pl.ds(nl, nl)] * wb)
                return 0

            lax.fori_loop(0, cnt // nl, _group, 0)

        def _run_chunks(chunks):
            edescs = []
            for (o, cnt) in chunks:
                edescs.append(pltpu.async_copy(
                    wbuf.at[pl.ds(o, cnt)], ews.at[sbuf.at[pl.ds(o, cnt)]],
                    esem, add=True))

            nch = len(chunks)
            gdescs = {}
            sdescs = {}

            def _issue_g(u):
                o, cnt = chunks[u]
                b = u % _NB
                dst = rows[b] if cnt == _CH else rows[b].at[pl.ds(0, cnt)]
                gdescs[u] = pltpu.async_copy(
                    probe_hbm.at[sbuf.at[pl.ds(o, cnt)]], dst, gsem[b])

            _issue_g(0)
            if nch > 1:
                _issue_g(1)
            for u in range(nch):
                o, cnt = chunks[u]
                b = u % _NB
                gdescs[u].wait()
                _scale(o, cnt, rows[b])
                src = rows[b] if cnt == _CH else rows[b].at[pl.ds(0, cnt)]
                sdescs[u] = pltpu.async_copy(
                    src, acc.at[dbuf.at[pl.ds(o, cnt)]], ssem[b], add=True)
                if u + 2 < nch:
                    if u - 2 >= 0:
                        sdescs[u - 2].wait()
                    _issue_g(u + 2)
                elif u - 2 >= 0:
                    sdescs[u - 2].wait()
            for d in edescs:
                d.wait()

        full_chunks = [(u * _CH, _CH) for u in range(_G)]

        def _drain_last_two(chunks):
            for u in (len(chunks) - 2, len(chunks) - 1):
                if u < 0:
                    continue
                o, cnt = chunks[u]
                b = u % _NB
                src = rows[b] if cnt == _CH else rows[b].at[pl.ds(0, cnt)]
                pltpu.make_async_copy(
                    src, acc.at[dbuf.at[pl.ds(o, cnt)]], ssem[b]).wait()

        def _block(t, _):
            @pl.when(t > 0)
            def _():
                _drain_last_two(full_chunks)

            off = wbase + t * bsz
            pltpu.sync_copy(ei_hbm.at[0, pl.ds(off, bsz)], sbuf)
            pltpu.sync_copy(ei_hbm.at[1, pl.ds(off, bsz)], dbuf)
            pltpu.sync_copy(w_hbm.at[pl.ds(off, bsz)], wbuf)
            _run_chunks(full_chunks)
            return 0

        lax.fori_loop(0, blk, _block, 0)
        _drain_last_two(full_chunks)
        if tail:
            toff = wbase + blk * bsz
            pltpu.sync_copy(ei_hbm.at[0, pl.ds(toff, tail)],
                            sbuf.at[pl.ds(0, tail)])
            pltpu.sync_copy(ei_hbm.at[1, pl.ds(toff, tail)],
                            dbuf.at[pl.ds(0, tail)])
            pltpu.sync_copy(w_hbm.at[pl.ds(toff, tail)],
                            wbuf.at[pl.ds(0, tail)])
            _run_chunks(tail_chunks)
            _drain_last_two(tail_chunks)
        plsc.subcore_barrier()

        pltpu.sync_copy(acc.at[pl.ds(sid * npt, npt)],
                        acc_hbm.at[cid, pl.ds(sid * npt, npt)])
        pltpu.sync_copy(ews.at[pl.ds(sid * npt, npt)],
                        ews_hbm.at[cid, pl.ds(sid * npt, npt)])

    return sc_scatter


def kernel(x, edge_index, edge_width, vertex_mass):
    n = x.shape[0]
    e = edge_index.shape[1]
    ei = edge_index.astype(jnp.int32)
    w1 = edge_width.astype(jnp.float32)[:, 0]

    wvec, freqs, amp = _probe_consts()
    f2 = jnp.concatenate([freqs, freqs], axis=1)
    f128 = jnp.tile(f2, (1, _RPR))
    a128 = jnp.tile(jnp.concatenate([amp, amp], axis=1), (1, _RPR))
    sel = (jnp.arange(128, dtype=jnp.int32) // _PC).reshape(1, 128)

    xx = x.reshape(n // _RPR, 3 * _RPR)
    probe128 = _make_probe(n)(xx, wvec, f128, a128, sel)

    accs, ews = _make_sc_scatter(n, e)(probe128.reshape(n, _PC), ei, w1)

    n_pad = ews.shape[1]
    acc128 = accs.reshape(2, n_pad // _RPR, 128)
    e4 = ews.reshape(2, n_pad // _RPR, _RPR)
    m4 = vertex_mass.reshape(n // _RPR, _RPR)
    out128 = _make_combine(n, n_pad)(acc128, acc128, e4, e4, probe128, m4, sel)
    return out128.reshape(n, _PC)
